# baseline (speedup 1.0000x reference)
import functools
import math

import jax
import jax.numpy as jnp
from jax import lax
from jax.experimental import pallas as pl
from jax.experimental.pallas import tpu as pltpu
from jax.experimental.pallas import tpu_sc as plsc

NC = 2
NS = 16
L = 16

AVG_NEIGHBORS = 32.0


def _make_attr_gather(n, e, ap):
    ce = 80
    epw = e // (NC * NS)
    nchunks = epw // ce
    assert epw % ce == 0 and ce % 8 == 0
    mesh = plsc.VectorSubcoreMesh(
        core_axis_name="c", subcore_axis_name="s", num_cores=NC, num_subcores=NS)

    @functools.partial(
        pl.kernel, mesh=mesh,
        out_type=jax.ShapeDtypeStruct((e, ap), jnp.float32),
        compiler_params=pltpu.CompilerParams(use_tc_tiling_on_sc=False),
        scratch_types=[
            pltpu.VMEM((ce,), jnp.int32),
            pltpu.VMEM((ce, ap), jnp.float32),
            pltpu.SemaphoreType.DMA,
        ],
    )
    def attr_gather(attr_hbm, sidx_hbm, out_hbm, sidx_v, rows_v, sem):
        c = lax.axis_index("c")
        s = lax.axis_index("s")
        base0 = (c * NS + s) * epw

        def chunk(j, carry):
            base = base0 + j * ce
            pltpu.sync_copy(sidx_hbm.at[pl.ds(base, ce)], sidx_v)
            pltpu.async_copy(attr_hbm.at[sidx_v], rows_v, sem).wait()
            pltpu.sync_copy(rows_v, out_hbm.at[pl.ds(base, ce), :])
            return carry

        lax.fori_loop(0, nchunks, chunk, 0)

    return attr_gather


def _make_edge_pass(n, e, d):
    ce = 80
    eps = e // NS
    nchunks = eps // ce
    assert eps % ce == 0
    npad = 10240
    rows_per_tile = npad // NS
    zr = 128
    assert n <= npad and rows_per_tile % zr == 0
    nk = d // L
    mesh = plsc.VectorSubcoreMesh(
        core_axis_name="c", subcore_axis_name="s", num_cores=NC, num_subcores=NS)

    @functools.partial(
        pl.kernel, mesh=mesh,
        out_type=jax.ShapeDtypeStruct((NC, npad, d), jnp.float32),
        compiler_params=pltpu.CompilerParams(needs_layout_passes=False),
        scratch_types=[
            pltpu.VMEM((ce,), jnp.int32),
            pltpu.VMEM((ce,), jnp.int32),
            pltpu.VMEM((ce,), jnp.float32),
            pltpu.VMEM((ce,), jnp.float32),
            pltpu.VMEM((ce,), jnp.float32),
            pltpu.VMEM((ce, d), jnp.float32),
            pltpu.VMEM((ce, d), jnp.float32),
            pltpu.VMEM((ce, d), jnp.float32),
            pltpu.VMEM((zr, d), jnp.float32),
            pltpu.VMEM_SHARED((npad, d), jnp.float32),
            pltpu.SemaphoreType.DMA,
        ],
    )
    def edge_pass(nf_hbm, tpw_hbm, ear_hbm, eai_hbm, sidx_hbm, ridx_hbm,
                  out_hbm, sidx_v, ridx_v, ear_v, eai_v, eam_v, tpw_v, nfr_v,
                  out_v, zero_v, acc_sh, sem):
        c = lax.axis_index("c")
        s = lax.axis_index("s")

        def zfill(r, carry):
            for k in range(nk):
                zero_v[r, pl.ds(k * L, L)] = jnp.zeros((L,), jnp.float32)
            return carry
        lax.fori_loop(0, zr, zfill, 0)

        def zcopy(t, carry):
            pltpu.sync_copy(
                zero_v, acc_sh.at[pl.ds(s * rows_per_tile + t * zr, zr), :])
            return carry
        lax.fori_loop(0, rows_per_tile // zr, zcopy, 0)
        plsc.subcore_barrier()

        is_real = lax.broadcast(c == 0, (L,))
        base0 = s * eps

        def chunk(j, carry):
            base = base0 + j * ce
            pltpu.sync_copy(sidx_hbm.at[pl.ds(base, ce)], sidx_v)
            pltpu.sync_copy(ridx_hbm.at[pl.ds(base, ce)], ridx_v)
            pltpu.sync_copy(ear_hbm.at[pl.ds(base, ce)], ear_v)
            pltpu.sync_copy(eai_hbm.at[pl.ds(base, ce)], eai_v)
            pltpu.sync_copy(tpw_hbm.at[pl.ds(base, ce), :], tpw_v)
            pltpu.async_copy(nf_hbm.at[sidx_v], nfr_v, sem).wait()
            for k in range(ce // L):
                sl = pl.ds(k * L, L)
                eam_v[sl] = jnp.where(is_real, ear_v[sl], eai_v[sl])

            def edge(ei, carry2):
                eav = plsc.load_gather(eam_v, [lax.broadcast(ei, (L,))])
                for k in range(nk):
                    sl = pl.ds(k * L, L)
                    out_v[ei, sl] = nfr_v[ei, sl] * tpw_v[ei, sl] * eav
                return carry2
            lax.fori_loop(0, ce, edge, 0)
            pltpu.sync_copy(out_v, acc_sh.at[ridx_v], add=True)
            return carry

        lax.fori_loop(0, nchunks, chunk, 0)
        plsc.subcore_barrier()

        def flush(t, carry):
            r0 = s * rows_per_tile + t * zr
            pltpu.sync_copy(
                acc_sh.at[pl.ds(r0, zr), :], out_hbm.at[c, pl.ds(r0, zr), :])
            return carry
        lax.fori_loop(0, rows_per_tile // zr, flush, 0)

    return edge_pass


def _make_tc_pre(n, d, a):
    bn = 1000
    assert n % bn == 0

    def body(feats_ref, attr_ref, wup_ref, wskip_ref, nf_ref, sc_ref):
        f = feats_ref[...]
        at = attr_ref[...]
        nf_ref[...] = jnp.dot(
            f, wup_ref[...], precision=lax.Precision.HIGHEST,
            preferred_element_type=jnp.float32) * (1.0 / math.sqrt(d))
        acc = jnp.zeros((bn, d), jnp.float32)
        for ai in range(a):
            acc += at[:, ai:ai + 1] * jnp.dot(
                f, wskip_ref[ai], precision=lax.Precision.HIGHEST,
                preferred_element_type=jnp.float32)
        sc_ref[...] = acc * (1.0 / math.sqrt(float(d * a)))

    return pl.pallas_call(
        body,
        grid=(n // bn,),
        in_specs=[
            pl.BlockSpec((bn, d), lambda i: (i, 0)),
            pl.BlockSpec((bn, 16), lambda i: (i, 0)),
            pl.BlockSpec((d, d), lambda i: (0, 0)),
            pl.BlockSpec((a, d, d), lambda i: (0, 0, 0)),
        ],
        out_specs=[
            pl.BlockSpec((bn, d), lambda i: (i, 0)),
            pl.BlockSpec((bn, d), lambda i: (i, 0)),
        ],
        out_shape=[
            jax.ShapeDtypeStruct((n, d), jnp.float32),
            jax.ShapeDtypeStruct((n, d), jnp.float32),
        ],
    )


def _make_tc_tpw(e, d, a, r):
    be = 2000
    assert e % be == 0

    def body(ef_ref, attrs_ref, tpw_ref, out_ref):
        ef = ef_ref[...]
        at = attrs_ref[...]
        acc = jnp.zeros((be, d), jnp.float32)
        for ai in range(a):
            acc += at[:, ai:ai + 1] * jnp.dot(
                ef, tpw_ref[ai], precision=lax.Precision.HIGHEST,
                preferred_element_type=jnp.float32)
        out_ref[...] = acc

    return pl.pallas_call(
        body,
        grid=(e // be,),
        in_specs=[
            pl.BlockSpec((be, r), lambda i: (i, 0)),
            pl.BlockSpec((be, 16), lambda i: (i, 0)),
            pl.BlockSpec((a, r, d), lambda i: (0, 0, 0)),
        ],
        out_specs=pl.BlockSpec((be, d), lambda i: (i, 0)),
        out_shape=jax.ShapeDtypeStruct((e, d), jnp.float32),
    )


def _make_tc_fin(n, d):
    bn = 1000
    cst = 1.0 / math.sqrt(float(d)) / AVG_NEIGHBORS

    def body(pre_ref, wlin_ref, outr_ref, outi_ref):
        w = wlin_ref[...]
        outr_ref[...] = jnp.dot(
            pre_ref[0], w, precision=lax.Precision.HIGHEST,
            preferred_element_type=jnp.float32) * cst
        outi_ref[...] = jnp.dot(
            pre_ref[1], w, precision=lax.Precision.HIGHEST,
            preferred_element_type=jnp.float32) * cst

    return pl.pallas_call(
        body,
        grid=(n // bn,),
        in_specs=[
            pl.BlockSpec((2, bn, d), lambda i: (0, i, 0)),
            pl.BlockSpec((d, d), lambda i: (0, 0)),
        ],
        out_specs=[
            pl.BlockSpec((bn, d), lambda i: (i, 0)),
            pl.BlockSpec((bn, d), lambda i: (i, 0)),
        ],
        out_shape=[
            jax.ShapeDtypeStruct((n, d), jnp.float32),
            jax.ShapeDtypeStruct((n, d), jnp.float32),
        ],
    )


def kernel(node_attrs, node_feats, edge_attrs, edge_feats, edge_index,
           W_up, tpw, W_lin, W_skip):
    n, a = node_attrs.shape
    d = node_feats.shape[1]
    e, r = edge_feats.shape

    sender = edge_index[0].astype(jnp.int32)
    receiver = edge_index[1].astype(jnp.int32)
    ea_r = edge_attrs[:, 0]
    ea_i = edge_attrs[:, 1]
    attr16 = jnp.concatenate(
        [node_attrs, jnp.zeros((n, 16 - a), jnp.float32)], axis=1)
    wskip_t = W_skip.transpose(1, 0, 2)

    attr_s = _make_attr_gather(n, e, 16)(attr16, sender)
    nf, sc = _make_tc_pre(n, d, a)(node_feats, attr16, W_up, wskip_t)
    tp_weights = _make_tc_tpw(e, d, a, r)(edge_feats, attr_s, tpw)
    partial = _make_edge_pass(n, e, d)(
        nf, tp_weights, ea_r, ea_i, sender, receiver)
    real, imag = _make_tc_fin(n, d)(partial[:, :n, :], W_lin)
    message = jnp.stack((real, imag), axis=-1).reshape(n, d, 1, 2)
    return (message, sc)


# speedup vs baseline: 1.7692x; 1.7692x over previous
import functools
import math

import jax
import jax.numpy as jnp
from jax import lax
from jax.experimental import pallas as pl
from jax.experimental.pallas import tpu as pltpu
from jax.experimental.pallas import tpu_sc as plsc

NC = 2
NS = 16
L = 16

AVG_NEIGHBORS = 32.0


def _make_attr_gather(n, e, ap):
    ce = 80
    epw = e // (NC * NS)
    nchunks = epw // ce
    assert epw % ce == 0 and ce % 8 == 0
    mesh = plsc.VectorSubcoreMesh(
        core_axis_name="c", subcore_axis_name="s", num_cores=NC, num_subcores=NS)

    @functools.partial(
        pl.kernel, mesh=mesh,
        out_type=jax.ShapeDtypeStruct((e, ap), jnp.float32),
        compiler_params=pltpu.CompilerParams(use_tc_tiling_on_sc=False),
        scratch_types=[
            pltpu.VMEM((ce,), jnp.int32),
            pltpu.VMEM((ce, ap), jnp.float32),
            pltpu.SemaphoreType.DMA,
        ],
    )
    def attr_gather(attr_hbm, sidx_hbm, out_hbm, sidx_v, rows_v, sem):
        c = lax.axis_index("c")
        s = lax.axis_index("s")
        base0 = (c * NS + s) * epw

        def chunk(j, carry):
            base = base0 + j * ce
            pltpu.sync_copy(sidx_hbm.at[pl.ds(base, ce)], sidx_v)
            pltpu.async_copy(attr_hbm.at[sidx_v], rows_v, sem).wait()
            pltpu.sync_copy(rows_v, out_hbm.at[pl.ds(base, ce), :])
            return carry

        lax.fori_loop(0, nchunks, chunk, 0)

    return attr_gather


def _make_edge_pass(n, e, d):
    ce = 40
    ring = 4
    eps = e // NS
    nchunks = eps // ce
    assert eps % ce == 0 and nchunks % ring == 0 and nchunks >= 2 * ring
    npad = 10240
    rows_per_tile = npad // NS
    assert n <= npad and rows_per_tile % ce == 0
    nk = d // L
    mesh = plsc.VectorSubcoreMesh(
        core_axis_name="c", subcore_axis_name="s", num_cores=NC, num_subcores=NS)

    scratch = (
        [pltpu.VMEM((ce,), jnp.int32)] * ring
        + [pltpu.VMEM((ce,), jnp.int32)] * ring
        + [pltpu.VMEM((ce, d), jnp.float32)] * ring
        + [pltpu.VMEM((ce, d), jnp.float32)] * ring
        + [pltpu.VMEM_SHARED((npad, d), jnp.float32)]
        + [pltpu.SemaphoreType.DMA] * (4 * ring)
    )

    @functools.partial(
        pl.kernel, mesh=mesh,
        out_type=jax.ShapeDtypeStruct((NC, npad, d), jnp.float32),
        compiler_params=pltpu.CompilerParams(needs_layout_passes=False),
        scratch_types=scratch,
    )
    def edge_pass(nf_hbm, tpw2_hbm, sidx_hbm, ridx_hbm, out_hbm, *sc):
        sidx_v = sc[0:ring]
        ridx_v = sc[ring:2 * ring]
        tpw_v = sc[2 * ring:3 * ring]
        nfr_v = sc[3 * ring:4 * ring]
        acc_sh = sc[4 * ring]
        sem_idx = sc[4 * ring + 1:4 * ring + 1 + ring]
        sem_tpw = sc[4 * ring + 1 + ring:4 * ring + 1 + 2 * ring]
        sem_g = sc[4 * ring + 1 + 2 * ring:4 * ring + 1 + 3 * ring]
        sem_sc = sc[4 * ring + 1 + 3 * ring:4 * ring + 1 + 4 * ring]
        c = lax.axis_index("c")
        s = lax.axis_index("s")

        def zfill(rr, carry):
            for k in range(nk):
                nfr_v[0][rr, pl.ds(k * L, L)] = jnp.zeros((L,), jnp.float32)
            return carry
        lax.fori_loop(0, ce, zfill, 0)

        def zcopy(t, carry):
            pltpu.sync_copy(
                nfr_v[0], acc_sh.at[pl.ds(s * rows_per_tile + t * ce, ce), :])
            return carry
        lax.fori_loop(0, rows_per_tile // ce, zcopy, 0)
        plsc.subcore_barrier()

        base0 = s * eps

        def start_idx(j, slot):
            base = base0 + j * ce
            pltpu.async_copy(sidx_hbm.at[pl.ds(base, ce)], sidx_v[slot],
                             sem_idx[slot])
            pltpu.async_copy(ridx_hbm.at[pl.ds(base, ce)], ridx_v[slot],
                             sem_idx[slot])

        def wait_idx(slot):
            pltpu.make_async_copy(sidx_hbm.at[pl.ds(0, ce)], sidx_v[slot],
                                  sem_idx[slot]).wait()
            pltpu.make_async_copy(ridx_hbm.at[pl.ds(0, ce)], ridx_v[slot],
                                  sem_idx[slot]).wait()

        def start_data(j, slot):
            base = base0 + j * ce
            pltpu.async_copy(tpw2_hbm.at[c, pl.ds(base, ce), :], tpw_v[slot],
                             sem_tpw[slot])
            pltpu.async_copy(nf_hbm.at[sidx_v[slot]], nfr_v[slot], sem_g[slot])

        def wait_data(slot):
            pltpu.make_async_copy(tpw2_hbm.at[0, pl.ds(0, ce), :], tpw_v[slot],
                                  sem_tpw[slot]).wait()
            pltpu.make_async_copy(nf_hbm.at[sidx_v[slot]], nfr_v[slot],
                                  sem_g[slot]).wait()

        def start_scatter(slot):
            pltpu.async_copy(tpw_v[slot], acc_sh.at[ridx_v[slot]],
                             sem_sc[slot], add=True)

        def wait_scatter(slot):
            pltpu.make_async_copy(tpw_v[slot], acc_sh.at[ridx_v[slot]],
                                  sem_sc[slot]).wait()

        start_idx(0, 0)
        start_idx(1, 1)
        wait_idx(0)
        start_data(0, 0)

        def chunk(j5, carry):
            for b in range(ring):
                j = j5 * ring + b
                @pl.when(jnp.logical_and(j >= ring - 2, j <= nchunks - 3))
                def _():
                    wait_scatter((b + 2) % ring)
                @pl.when(j <= nchunks - 3)
                def _():
                    start_idx(j + 2, (b + 2) % ring)
                @pl.when(j <= nchunks - 2)
                def _():
                    wait_idx((b + 1) % ring)
                    start_data(j + 1, (b + 1) % ring)
                wait_data(b)

                def edge(ei, carry2):
                    for k in range(nk):
                        sl = pl.ds(k * L, L)
                        tpw_v[b][ei, sl] = tpw_v[b][ei, sl] * nfr_v[b][ei, sl]
                    return carry2
                lax.fori_loop(0, ce, edge, 0)
                start_scatter(b)
            return carry

        lax.fori_loop(0, nchunks // ring, chunk, 0)
        for jj in range(nchunks - ring, nchunks):
            wait_scatter(jj % ring)
        plsc.subcore_barrier()

        def flush(t, carry):
            r0 = s * rows_per_tile + t * ce
            pltpu.sync_copy(
                acc_sh.at[pl.ds(r0, ce), :], out_hbm.at[c, pl.ds(r0, ce), :])
            return carry
        lax.fori_loop(0, rows_per_tile // ce, flush, 0)

    return edge_pass


def _make_tc_pre(n, d, a):
    bn = 1000
    assert n % bn == 0

    def body(feats_ref, attr_ref, wup_ref, wskip_ref, nf_ref, sc_ref):
        f = feats_ref[...]
        at = attr_ref[...]
        nf_ref[...] = jnp.dot(
            f, wup_ref[...], precision=lax.Precision.HIGHEST,
            preferred_element_type=jnp.float32) * (1.0 / math.sqrt(d))
        acc = jnp.zeros((bn, d), jnp.float32)
        for ai in range(a):
            acc += at[:, ai:ai + 1] * jnp.dot(
                f, wskip_ref[ai], precision=lax.Precision.HIGHEST,
                preferred_element_type=jnp.float32)
        sc_ref[...] = acc * (1.0 / math.sqrt(float(d * a)))

    return pl.pallas_call(
        body,
        grid=(n // bn,),
        in_specs=[
            pl.BlockSpec((bn, d), lambda i: (i, 0)),
            pl.BlockSpec((bn, 16), lambda i: (i, 0)),
            pl.BlockSpec((d, d), lambda i: (0, 0)),
            pl.BlockSpec((a, d, d), lambda i: (0, 0, 0)),
        ],
        out_specs=[
            pl.BlockSpec((bn, d), lambda i: (i, 0)),
            pl.BlockSpec((bn, d), lambda i: (i, 0)),
        ],
        out_shape=[
            jax.ShapeDtypeStruct((n, d), jnp.float32),
            jax.ShapeDtypeStruct((n, d), jnp.float32),
        ],
    )


def _make_tc_tpw(e, d, a, r):
    be = 2000
    assert e % be == 0

    def body(ef_ref, attrs_ref, tpw_ref, ea_ref, out_ref):
        ef = ef_ref[...]
        at = attrs_ref[...]
        acc = jnp.zeros((be, d), jnp.float32)
        for ai in range(a):
            acc += at[:, ai:ai + 1] * jnp.dot(
                ef, tpw_ref[ai], precision=lax.Precision.HIGHEST,
                preferred_element_type=jnp.float32)
        ea = ea_ref[...]
        out_ref[0] = acc * ea[:, 0:1]
        out_ref[1] = acc * ea[:, 1:2]

    return pl.pallas_call(
        body,
        grid=(e // be,),
        in_specs=[
            pl.BlockSpec((be, r), lambda i: (i, 0)),
            pl.BlockSpec((be, 16), lambda i: (i, 0)),
            pl.BlockSpec((a, r, d), lambda i: (0, 0, 0)),
            pl.BlockSpec((be, 2), lambda i: (i, 0)),
        ],
        out_specs=pl.BlockSpec((2, be, d), lambda i: (0, i, 0)),
        out_shape=jax.ShapeDtypeStruct((2, e, d), jnp.float32),
    )


def _make_tc_fin(n, d):
    bn = 1000
    cst = 1.0 / math.sqrt(float(d)) / AVG_NEIGHBORS

    def body(pre_ref, wlin_ref, outr_ref, outi_ref):
        w = wlin_ref[...]
        outr_ref[...] = jnp.dot(
            pre_ref[0], w, precision=lax.Precision.HIGHEST,
            preferred_element_type=jnp.float32) * cst
        outi_ref[...] = jnp.dot(
            pre_ref[1], w, precision=lax.Precision.HIGHEST,
            preferred_element_type=jnp.float32) * cst

    return pl.pallas_call(
        body,
        grid=(n // bn,),
        in_specs=[
            pl.BlockSpec((2, bn, d), lambda i: (0, i, 0)),
            pl.BlockSpec((d, d), lambda i: (0, 0)),
        ],
        out_specs=[
            pl.BlockSpec((bn, d), lambda i: (i, 0)),
            pl.BlockSpec((bn, d), lambda i: (i, 0)),
        ],
        out_shape=[
            jax.ShapeDtypeStruct((n, d), jnp.float32),
            jax.ShapeDtypeStruct((n, d), jnp.float32),
        ],
    )


def kernel(node_attrs, node_feats, edge_attrs, edge_feats, edge_index,
           W_up, tpw, W_lin, W_skip):
    n, a = node_attrs.shape
    d = node_feats.shape[1]
    e, r = edge_feats.shape

    sender = edge_index[0].astype(jnp.int32)
    receiver = edge_index[1].astype(jnp.int32)
    attr16 = jnp.concatenate(
        [node_attrs, jnp.zeros((n, 16 - a), jnp.float32)], axis=1)
    wskip_t = W_skip.transpose(1, 0, 2)

    attr_s = _make_attr_gather(n, e, 16)(attr16, sender)
    nf, sc = _make_tc_pre(n, d, a)(node_feats, attr16, W_up, wskip_t)
    tpw2 = _make_tc_tpw(e, d, a, r)(edge_feats, attr_s, tpw, edge_attrs)
    partial = _make_edge_pass(n, e, d)(nf, tpw2, sender, receiver)
    real, imag = _make_tc_fin(n, d)(partial[:, :n, :], W_lin)
    message = jnp.stack((real, imag), axis=-1).reshape(n, d, 1, 2)
    return (message, sc)


# speedup vs baseline: 2.4162x; 1.3657x over previous
import functools
import math

import jax
import jax.numpy as jnp
from jax import lax
from jax.experimental import pallas as pl
from jax.experimental.pallas import tpu as pltpu
from jax.experimental.pallas import tpu_sc as plsc

NC = 2
NS = 16
L = 16

AVG_NEIGHBORS = 32.0


def _make_attr_gather(n, e, ap):
    ce = 80
    ring = 4
    epw = e // (NC * NS)
    nchunks = epw // ce
    assert epw % ce == 0 and ce % 8 == 0
    niter = -(-nchunks // ring) * ring
    mesh = plsc.VectorSubcoreMesh(
        core_axis_name="c", subcore_axis_name="s", num_cores=NC, num_subcores=NS)

    scratch = (
        [pltpu.VMEM((ce,), jnp.int32)] * ring
        + [pltpu.VMEM((ce, ap), jnp.float32)] * ring
        + [pltpu.SemaphoreType.DMA] * (3 * ring)
    )

    @functools.partial(
        pl.kernel, mesh=mesh,
        out_type=jax.ShapeDtypeStruct((e, ap), jnp.float32),
        compiler_params=pltpu.CompilerParams(use_tc_tiling_on_sc=False),
        scratch_types=scratch,
    )
    def attr_gather(attr_hbm, sidx_hbm, out_hbm, *sc):
        sidx_v = sc[0:ring]
        rows_v = sc[ring:2 * ring]
        sem_idx = sc[2 * ring:3 * ring]
        sem_g = sc[3 * ring:4 * ring]
        sem_wb = sc[4 * ring:5 * ring]
        c = lax.axis_index("c")
        s = lax.axis_index("s")
        base0 = (c * NS + s) * epw

        def start_idx(j, slot):
            pltpu.async_copy(sidx_hbm.at[pl.ds(base0 + j * ce, ce)],
                             sidx_v[slot], sem_idx[slot])

        def wait_idx(slot):
            pltpu.make_async_copy(sidx_hbm.at[pl.ds(0, ce)], sidx_v[slot],
                                  sem_idx[slot]).wait()

        def start_gather(slot):
            pltpu.async_copy(attr_hbm.at[sidx_v[slot]], rows_v[slot],
                             sem_g[slot])

        def wait_gather(slot):
            pltpu.make_async_copy(attr_hbm.at[sidx_v[slot]], rows_v[slot],
                                  sem_g[slot]).wait()

        def start_wb(j, slot):
            pltpu.async_copy(rows_v[slot],
                             out_hbm.at[pl.ds(base0 + j * ce, ce), :],
                             sem_wb[slot])

        def wait_wb(slot):
            pltpu.make_async_copy(rows_v[slot],
                                  out_hbm.at[pl.ds(0, ce), :],
                                  sem_wb[slot]).wait()

        start_idx(0, 0)
        start_idx(1, 1)
        wait_idx(0)
        start_gather(0)

        def chunk(j4, carry):
            for b in range(ring):
                j = j4 * ring + b
                @pl.when(jnp.logical_and(j >= 2, j + 2 <= nchunks - 1))
                def _():
                    wait_wb((b + 2) % ring)
                @pl.when(j + 2 <= nchunks - 1)
                def _():
                    start_idx(j + 2, (b + 2) % ring)
                @pl.when(j + 1 <= nchunks - 1)
                def _():
                    wait_idx((b + 1) % ring)
                    start_gather((b + 1) % ring)
                @pl.when(j <= nchunks - 1)
                def _():
                    wait_gather(b)
                    start_wb(j, b)
            return carry

        lax.fori_loop(0, niter // ring, chunk, 0)
        for jj in range(nchunks - ring, nchunks):
            wait_wb(jj % ring)

    return attr_gather


def _make_edge_pass(n, e, d):
    ce = 40
    ring = 4
    eps = e // NS
    nchunks = eps // ce
    assert eps % ce == 0 and nchunks % ring == 0 and nchunks >= 2 * ring
    npad = 10240
    rows_per_tile = npad // NS
    assert n <= npad and rows_per_tile % ce == 0
    nk = d // L
    mesh = plsc.VectorSubcoreMesh(
        core_axis_name="c", subcore_axis_name="s", num_cores=NC, num_subcores=NS)

    scratch = (
        [pltpu.VMEM((ce,), jnp.int32)] * ring
        + [pltpu.VMEM((ce,), jnp.int32)] * ring
        + [pltpu.VMEM((ce, d), jnp.float32)] * ring
        + [pltpu.VMEM((ce, d), jnp.float32)] * ring
        + [pltpu.VMEM_SHARED((npad, d), jnp.float32)]
        + [pltpu.SemaphoreType.DMA] * (4 * ring)
    )

    @functools.partial(
        pl.kernel, mesh=mesh,
        out_type=jax.ShapeDtypeStruct((NC, npad, d), jnp.float32),
        compiler_params=pltpu.CompilerParams(needs_layout_passes=False),
        scratch_types=scratch,
    )
    def edge_pass(nf_hbm, tpw2_hbm, sidx_hbm, ridx_hbm, out_hbm, *sc):
        sidx_v = sc[0:ring]
        ridx_v = sc[ring:2 * ring]
        tpw_v = sc[2 * ring:3 * ring]
        nfr_v = sc[3 * ring:4 * ring]
        acc_sh = sc[4 * ring]
        sem_idx = sc[4 * ring + 1:4 * ring + 1 + ring]
        sem_tpw = sc[4 * ring + 1 + ring:4 * ring + 1 + 2 * ring]
        sem_g = sc[4 * ring + 1 + 2 * ring:4 * ring + 1 + 3 * ring]
        sem_sc = sc[4 * ring + 1 + 3 * ring:4 * ring + 1 + 4 * ring]
        c = lax.axis_index("c")
        s = lax.axis_index("s")

        def zfill(rr, carry):
            for k in range(nk):
                nfr_v[0][rr, pl.ds(k * L, L)] = jnp.zeros((L,), jnp.float32)
            return carry
        lax.fori_loop(0, ce, zfill, 0)

        def zcopy(t, carry):
            pltpu.sync_copy(
                nfr_v[0], acc_sh.at[pl.ds(s * rows_per_tile + t * ce, ce), :])
            return carry
        lax.fori_loop(0, rows_per_tile // ce, zcopy, 0)
        plsc.subcore_barrier()

        base0 = s * eps

        def start_idx(j, slot):
            base = base0 + j * ce
            pltpu.async_copy(sidx_hbm.at[pl.ds(base, ce)], sidx_v[slot],
                             sem_idx[slot])
            pltpu.async_copy(ridx_hbm.at[pl.ds(base, ce)], ridx_v[slot],
                             sem_idx[slot])

        def wait_idx(slot):
            pltpu.make_async_copy(sidx_hbm.at[pl.ds(0, ce)], sidx_v[slot],
                                  sem_idx[slot]).wait()
            pltpu.make_async_copy(ridx_hbm.at[pl.ds(0, ce)], ridx_v[slot],
                                  sem_idx[slot]).wait()

        def start_data(j, slot):
            base = base0 + j * ce
            pltpu.async_copy(tpw2_hbm.at[c, pl.ds(base, ce), :], tpw_v[slot],
                             sem_tpw[slot])
            pltpu.async_copy(nf_hbm.at[sidx_v[slot]], nfr_v[slot], sem_g[slot])

        def wait_data(slot):
            pltpu.make_async_copy(tpw2_hbm.at[0, pl.ds(0, ce), :], tpw_v[slot],
                                  sem_tpw[slot]).wait()
            pltpu.make_async_copy(nf_hbm.at[sidx_v[slot]], nfr_v[slot],
                                  sem_g[slot]).wait()

        def start_scatter(slot):
            pltpu.async_copy(tpw_v[slot], acc_sh.at[ridx_v[slot]],
                             sem_sc[slot], add=True)

        def wait_scatter(slot):
            pltpu.make_async_copy(tpw_v[slot], acc_sh.at[ridx_v[slot]],
                                  sem_sc[slot]).wait()

        start_idx(0, 0)
        start_idx(1, 1)
        wait_idx(0)
        start_data(0, 0)

        def chunk(j5, carry):
            for b in range(ring):
                j = j5 * ring + b
                @pl.when(jnp.logical_and(j >= ring - 2, j <= nchunks - 3))
                def _():
                    wait_scatter((b + 2) % ring)
                @pl.when(j <= nchunks - 3)
                def _():
                    start_idx(j + 2, (b + 2) % ring)
                @pl.when(j <= nchunks - 2)
                def _():
                    wait_idx((b + 1) % ring)
                    start_data(j + 1, (b + 1) % ring)
                wait_data(b)

                def edge(ei, carry2):
                    for k in range(nk):
                        sl = pl.ds(k * L, L)
                        tpw_v[b][ei, sl] = tpw_v[b][ei, sl] * nfr_v[b][ei, sl]
                    return carry2
                lax.fori_loop(0, ce, edge, 0)
                start_scatter(b)
            return carry

        lax.fori_loop(0, nchunks // ring, chunk, 0)
        for jj in range(nchunks - ring, nchunks):
            wait_scatter(jj % ring)
        plsc.subcore_barrier()

        def flush(t, carry):
            r0 = s * rows_per_tile + t * ce
            pltpu.sync_copy(
                acc_sh.at[pl.ds(r0, ce), :], out_hbm.at[c, pl.ds(r0, ce), :])
            return carry
        lax.fori_loop(0, rows_per_tile // ce, flush, 0)

    return edge_pass


def _make_tc_pre(n, d, a):
    bn = 1000
    assert n % bn == 0

    def body(feats_ref, attr_ref, wup_ref, wskip_ref, nf_ref, sc_ref):
        f = feats_ref[...]
        at = attr_ref[...]
        nf_ref[...] = jnp.dot(
            f, wup_ref[...], precision=lax.Precision.HIGHEST,
            preferred_element_type=jnp.float32) * (1.0 / math.sqrt(d))
        acc = jnp.zeros((bn, d), jnp.float32)
        for ai in range(a):
            acc += at[:, ai:ai + 1] * jnp.dot(
                f, wskip_ref[ai], precision=lax.Precision.HIGHEST,
                preferred_element_type=jnp.float32)
        sc_ref[...] = acc * (1.0 / math.sqrt(float(d * a)))

    return pl.pallas_call(
        body,
        grid=(n // bn,),
        in_specs=[
            pl.BlockSpec((bn, d), lambda i: (i, 0)),
            pl.BlockSpec((bn, 16), lambda i: (i, 0)),
            pl.BlockSpec((d, d), lambda i: (0, 0)),
            pl.BlockSpec((a, d, d), lambda i: (0, 0, 0)),
        ],
        out_specs=[
            pl.BlockSpec((bn, d), lambda i: (i, 0)),
            pl.BlockSpec((bn, d), lambda i: (i, 0)),
        ],
        out_shape=[
            jax.ShapeDtypeStruct((n, d), jnp.float32),
            jax.ShapeDtypeStruct((n, d), jnp.float32),
        ],
    )


def _make_tc_tpw(e, d, a, r):
    be = 2000
    assert e % be == 0

    def body(ef_ref, attrs_ref, tpw_ref, ea_ref, out_ref):
        ef = ef_ref[...]
        at = attrs_ref[...]
        g = jnp.concatenate(
            [(at[:, ai:ai + 1] * ef).astype(jnp.bfloat16) for ai in range(a)],
            axis=1)
        acc = jnp.dot(g, tpw_ref[...], preferred_element_type=jnp.float32)
        ea = ea_ref[...]
        out_ref[0] = acc * ea[:, 0:1]
        out_ref[1] = acc * ea[:, 1:2]

    return pl.pallas_call(
        body,
        grid=(e // be,),
        in_specs=[
            pl.BlockSpec((be, r), lambda i: (i, 0)),
            pl.BlockSpec((be, 16), lambda i: (i, 0)),
            pl.BlockSpec((a * r, d), lambda i: (0, 0)),
            pl.BlockSpec((be, 2), lambda i: (i, 0)),
        ],
        out_specs=pl.BlockSpec((2, be, d), lambda i: (0, i, 0)),
        out_shape=jax.ShapeDtypeStruct((2, e, d), jnp.float32),
    )


def _make_tc_fin(n, d):
    bn = 1000
    cst = 1.0 / math.sqrt(float(d)) / AVG_NEIGHBORS

    def body(pre_ref, wlin_ref, outr_ref, outi_ref):
        w = wlin_ref[...]
        outr_ref[...] = jnp.dot(
            pre_ref[0], w, precision=lax.Precision.HIGHEST,
            preferred_element_type=jnp.float32) * cst
        outi_ref[...] = jnp.dot(
            pre_ref[1], w, precision=lax.Precision.HIGHEST,
            preferred_element_type=jnp.float32) * cst

    return pl.pallas_call(
        body,
        grid=(n // bn,),
        in_specs=[
            pl.BlockSpec((2, bn, d), lambda i: (0, i, 0)),
            pl.BlockSpec((d, d), lambda i: (0, 0)),
        ],
        out_specs=[
            pl.BlockSpec((bn, d), lambda i: (i, 0)),
            pl.BlockSpec((bn, d), lambda i: (i, 0)),
        ],
        out_shape=[
            jax.ShapeDtypeStruct((n, d), jnp.float32),
            jax.ShapeDtypeStruct((n, d), jnp.float32),
        ],
    )


def kernel(node_attrs, node_feats, edge_attrs, edge_feats, edge_index,
           W_up, tpw, W_lin, W_skip):
    n, a = node_attrs.shape
    d = node_feats.shape[1]
    e, r = edge_feats.shape

    sender = edge_index[0].astype(jnp.int32)
    receiver = edge_index[1].astype(jnp.int32)
    attr16 = jnp.concatenate(
        [node_attrs, jnp.zeros((n, 16 - a), jnp.float32)], axis=1)
    wskip_t = W_skip.transpose(1, 0, 2)

    attr_s = _make_attr_gather(n, e, 16)(attr16, sender)
    nf, sc = _make_tc_pre(n, d, a)(node_feats, attr16, W_up, wskip_t)
    tpw_flat16 = tpw.reshape(a * r, d).astype(jnp.bfloat16)
    tpw2 = _make_tc_tpw(e, d, a, r)(edge_feats, attr_s, tpw_flat16, edge_attrs)
    partial = _make_edge_pass(n, e, d)(nf, tpw2, sender, receiver)
    real, imag = _make_tc_fin(n, d)(partial[:, :n, :], W_lin)
    message = jnp.stack((real, imag), axis=-1).reshape(n, d, 1, 2)
    return (message, sc)


# speedup vs baseline: 2.8996x; 1.2001x over previous
import functools
import math

import jax
import jax.numpy as jnp
from jax import lax
from jax.experimental import pallas as pl
from jax.experimental.pallas import tpu as pltpu
from jax.experimental.pallas import tpu_sc as plsc

NC = 2
NS = 16
L = 16

AVG_NEIGHBORS = 32.0


def _make_attr_gather(n, e, ap):
    ce = 80
    ring = 4
    epw = e // (NC * NS)
    nchunks = epw // ce
    assert epw % ce == 0 and ce % 8 == 0
    niter = -(-nchunks // ring) * ring
    mesh = plsc.VectorSubcoreMesh(
        core_axis_name="c", subcore_axis_name="s", num_cores=NC, num_subcores=NS)

    scratch = (
        [pltpu.VMEM((ce,), jnp.int32)] * ring
        + [pltpu.VMEM((ce, ap), jnp.float32)] * ring
        + [pltpu.SemaphoreType.DMA] * (3 * ring)
    )

    @functools.partial(
        pl.kernel, mesh=mesh,
        out_type=jax.ShapeDtypeStruct((e, ap), jnp.float32),
        compiler_params=pltpu.CompilerParams(use_tc_tiling_on_sc=False),
        scratch_types=scratch,
    )
    def attr_gather(attr_hbm, sidx_hbm, out_hbm, *sc):
        sidx_v = sc[0:ring]
        rows_v = sc[ring:2 * ring]
        sem_idx = sc[2 * ring:3 * ring]
        sem_g = sc[3 * ring:4 * ring]
        sem_wb = sc[4 * ring:5 * ring]
        c = lax.axis_index("c")
        s = lax.axis_index("s")
        base0 = (c * NS + s) * epw

        def start_idx(j, slot):
            pltpu.async_copy(sidx_hbm.at[pl.ds(base0 + j * ce, ce)],
                             sidx_v[slot], sem_idx[slot])

        def wait_idx(slot):
            pltpu.make_async_copy(sidx_hbm.at[pl.ds(0, ce)], sidx_v[slot],
                                  sem_idx[slot]).wait()

        def start_gather(slot):
            pltpu.async_copy(attr_hbm.at[sidx_v[slot]], rows_v[slot],
                             sem_g[slot])

        def wait_gather(slot):
            pltpu.make_async_copy(attr_hbm.at[sidx_v[slot]], rows_v[slot],
                                  sem_g[slot]).wait()

        def start_wb(j, slot):
            pltpu.async_copy(rows_v[slot],
                             out_hbm.at[pl.ds(base0 + j * ce, ce), :],
                             sem_wb[slot])

        def wait_wb(slot):
            pltpu.make_async_copy(rows_v[slot],
                                  out_hbm.at[pl.ds(0, ce), :],
                                  sem_wb[slot]).wait()

        start_idx(0, 0)
        start_idx(1, 1)
        wait_idx(0)
        start_gather(0)

        def chunk(j4, carry):
            for b in range(ring):
                j = j4 * ring + b
                @pl.when(jnp.logical_and(j >= 2, j + 2 <= nchunks - 1))
                def _():
                    wait_wb((b + 2) % ring)
                @pl.when(j + 2 <= nchunks - 1)
                def _():
                    start_idx(j + 2, (b + 2) % ring)
                @pl.when(j + 1 <= nchunks - 1)
                def _():
                    wait_idx((b + 1) % ring)
                    start_gather((b + 1) % ring)
                @pl.when(j <= nchunks - 1)
                def _():
                    wait_gather(b)
                    start_wb(j, b)
            return carry

        lax.fori_loop(0, niter // ring, chunk, 0)
        for jj in range(nchunks - ring, nchunks):
            wait_wb(jj % ring)

    return attr_gather


def _make_edge_pass(n, e, d):
    ce = 40
    ring = 4
    eps = e // NS
    nchunks = eps // ce
    assert eps % ce == 0 and nchunks % ring == 0 and nchunks >= 2 * ring
    npad = 10240
    rows_per_tile = npad // NS
    assert n <= npad and rows_per_tile % ce == 0
    nk = d // L
    mesh = plsc.VectorSubcoreMesh(
        core_axis_name="c", subcore_axis_name="s", num_cores=NC, num_subcores=NS)

    scratch = (
        [pltpu.VMEM((ce,), jnp.int32)] * ring
        + [pltpu.VMEM((ce,), jnp.int32)] * ring
        + [pltpu.VMEM((ce, d), jnp.float32)] * ring
        + [pltpu.VMEM((ce, d), jnp.float32)] * ring
        + [pltpu.VMEM_SHARED((npad, d), jnp.float32)]
        + [pltpu.SemaphoreType.DMA] * (4 * ring)
    )

    @functools.partial(
        pl.kernel, mesh=mesh,
        out_type=jax.ShapeDtypeStruct((NC, npad, d), jnp.float32),
        compiler_params=pltpu.CompilerParams(needs_layout_passes=False),
        scratch_types=scratch,
    )
    def edge_pass(nf_hbm, tpw2_hbm, sidx_hbm, ridx_hbm, out_hbm, *sc):
        sidx_v = sc[0:ring]
        ridx_v = sc[ring:2 * ring]
        tpw_v = sc[2 * ring:3 * ring]
        nfr_v = sc[3 * ring:4 * ring]
        acc_sh = sc[4 * ring]
        sem_idx = sc[4 * ring + 1:4 * ring + 1 + ring]
        sem_tpw = sc[4 * ring + 1 + ring:4 * ring + 1 + 2 * ring]
        sem_g = sc[4 * ring + 1 + 2 * ring:4 * ring + 1 + 3 * ring]
        sem_sc = sc[4 * ring + 1 + 3 * ring:4 * ring + 1 + 4 * ring]
        c = lax.axis_index("c")
        s = lax.axis_index("s")

        def zfill(rr, carry):
            for k in range(nk):
                tpw_v[0][rr, pl.ds(k * L, L)] = jnp.zeros((L,), jnp.float32)
            return carry
        lax.fori_loop(0, ce, zfill, 0)

        def zcopy(t, carry):
            pltpu.sync_copy(
                tpw_v[0], acc_sh.at[pl.ds(s * rows_per_tile + t * ce, ce), :])
            return carry
        lax.fori_loop(0, rows_per_tile // ce, zcopy, 0)
        plsc.subcore_barrier()

        base0 = s * eps

        def start_idx(j, slot):
            base = base0 + j * ce
            pltpu.async_copy(sidx_hbm.at[pl.ds(base, ce)], sidx_v[slot],
                             sem_idx[slot])
            pltpu.async_copy(ridx_hbm.at[pl.ds(base, ce)], ridx_v[slot],
                             sem_idx[slot])

        def wait_idx(slot):
            pltpu.make_async_copy(sidx_hbm.at[pl.ds(0, ce)], sidx_v[slot],
                                  sem_idx[slot]).wait()
            pltpu.make_async_copy(ridx_hbm.at[pl.ds(0, ce)], ridx_v[slot],
                                  sem_idx[slot]).wait()

        def start_data(j, slot):
            base = base0 + j * ce
            pltpu.async_copy(tpw2_hbm.at[c, pl.ds(base, ce), :], tpw_v[slot],
                             sem_tpw[slot])
            pltpu.async_copy(nf_hbm.at[sidx_v[slot]], nfr_v[slot], sem_g[slot])

        def wait_data(slot):
            pltpu.make_async_copy(tpw2_hbm.at[0, pl.ds(0, ce), :], tpw_v[slot],
                                  sem_tpw[slot]).wait()
            pltpu.make_async_copy(nf_hbm.at[sidx_v[slot]], nfr_v[slot],
                                  sem_g[slot]).wait()

        def start_scatter(slot):
            pltpu.async_copy(tpw_v[slot], acc_sh.at[ridx_v[slot]],
                             sem_sc[slot], add=True)

        def wait_scatter(slot):
            pltpu.make_async_copy(tpw_v[slot], acc_sh.at[ridx_v[slot]],
                                  sem_sc[slot]).wait()

        start_idx(0, 0)
        start_idx(1, 1)
        wait_idx(0)
        start_data(0, 0)

        def chunk(j5, carry):
            for b in range(ring):
                j = j5 * ring + b
                @pl.when(jnp.logical_and(j >= ring - 2, j <= nchunks - 3))
                def _():
                    wait_scatter((b + 2) % ring)
                @pl.when(j <= nchunks - 3)
                def _():
                    start_idx(j + 2, (b + 2) % ring)
                @pl.when(j <= nchunks - 2)
                def _():
                    wait_idx((b + 1) % ring)
                    start_data(j + 1, (b + 1) % ring)
                wait_data(b)

                def edge(ei, carry2):
                    for k in range(nk):
                        sl = pl.ds(k * L, L)
                        tpw_v[b][ei, sl] = tpw_v[b][ei, sl] * nfr_v[b][ei, sl]
                    return carry2
                lax.fori_loop(0, ce, edge, 0)
                start_scatter(b)
            return carry

        lax.fori_loop(0, nchunks // ring, chunk, 0)
        for jj in range(nchunks - ring, nchunks):
            wait_scatter(jj % ring)
        plsc.subcore_barrier()

        def flush(t, carry):
            r0 = s * rows_per_tile + t * ce
            pltpu.sync_copy(
                acc_sh.at[pl.ds(r0, ce), :], out_hbm.at[c, pl.ds(r0, ce), :])
            return carry
        lax.fori_loop(0, rows_per_tile // ce, flush, 0)

    return edge_pass


def _make_tc_pre(n, d, a):
    bn = 1000
    assert n % bn == 0

    def body(feats_ref, attr_ref, wup_ref, wskip_ref, nf_ref, sc_ref):
        f = feats_ref[...]
        at = attr_ref[...]
        nf_ref[...] = jnp.dot(
            f, wup_ref[...], precision=lax.Precision.HIGHEST,
            preferred_element_type=jnp.float32) * (1.0 / math.sqrt(d))
        acc = jnp.zeros((bn, d), jnp.float32)
        for ai in range(a):
            acc += at[:, ai:ai + 1] * jnp.dot(
                f, wskip_ref[ai], precision=lax.Precision.HIGHEST,
                preferred_element_type=jnp.float32)
        sc_ref[...] = acc * (1.0 / math.sqrt(float(d * a)))

    return pl.pallas_call(
        body,
        grid=(n // bn,),
        in_specs=[
            pl.BlockSpec((bn, d), lambda i: (i, 0)),
            pl.BlockSpec((bn, 16), lambda i: (i, 0)),
            pl.BlockSpec((d, d), lambda i: (0, 0)),
            pl.BlockSpec((a, d, d), lambda i: (0, 0, 0)),
        ],
        out_specs=[
            pl.BlockSpec((bn, d), lambda i: (i, 0)),
            pl.BlockSpec((bn, d), lambda i: (i, 0)),
        ],
        out_shape=[
            jax.ShapeDtypeStruct((n, d), jnp.float32),
            jax.ShapeDtypeStruct((n, d), jnp.float32),
        ],
    )


def _make_tc_tpw(e, d, a, r):
    be = 2000
    assert e % be == 0

    def body(ef_ref, attrs_ref, tpw_ref, ea_ref, out_ref):
        ef = ef_ref[...]
        at = attrs_ref[...]
        z = jnp.dot(at[:, :a].astype(jnp.bfloat16), tpw_ref[...],
                    preferred_element_type=jnp.float32)
        acc = jnp.zeros((be, d), jnp.float32)
        for ri in range(r):
            acc += ef[:, ri:ri + 1] * z[:, ri * d:(ri + 1) * d]
        ea = ea_ref[...]
        out_ref[0] = acc * ea[:, 0:1]
        out_ref[1] = acc * ea[:, 1:2]

    return pl.pallas_call(
        body,
        grid=(e // be,),
        in_specs=[
            pl.BlockSpec((be, r), lambda i: (i, 0)),
            pl.BlockSpec((be, 16), lambda i: (i, 0)),
            pl.BlockSpec((a, r * d), lambda i: (0, 0)),
            pl.BlockSpec((be, 2), lambda i: (i, 0)),
        ],
        out_specs=pl.BlockSpec((2, be, d), lambda i: (0, i, 0)),
        out_shape=jax.ShapeDtypeStruct((2, e, d), jnp.float32),
    )


def _make_tc_fin(n, d):
    bn = 1000
    cst = 1.0 / math.sqrt(float(d)) / AVG_NEIGHBORS

    def body(pre_ref, wlin_ref, outr_ref, outi_ref):
        w = wlin_ref[...]
        outr_ref[...] = jnp.dot(
            pre_ref[0], w, precision=lax.Precision.HIGHEST,
            preferred_element_type=jnp.float32) * cst
        outi_ref[...] = jnp.dot(
            pre_ref[1], w, precision=lax.Precision.HIGHEST,
            preferred_element_type=jnp.float32) * cst

    return pl.pallas_call(
        body,
        grid=(n // bn,),
        in_specs=[
            pl.BlockSpec((2, bn, d), lambda i: (0, i, 0)),
            pl.BlockSpec((d, d), lambda i: (0, 0)),
        ],
        out_specs=[
            pl.BlockSpec((bn, d), lambda i: (i, 0)),
            pl.BlockSpec((bn, d), lambda i: (i, 0)),
        ],
        out_shape=[
            jax.ShapeDtypeStruct((n, d), jnp.float32),
            jax.ShapeDtypeStruct((n, d), jnp.float32),
        ],
    )


def kernel(node_attrs, node_feats, edge_attrs, edge_feats, edge_index,
           W_up, tpw, W_lin, W_skip):
    n, a = node_attrs.shape
    d = node_feats.shape[1]
    e, r = edge_feats.shape

    sender = edge_index[0].astype(jnp.int32)
    receiver = edge_index[1].astype(jnp.int32)
    attr16 = jnp.concatenate(
        [node_attrs, jnp.zeros((n, 16 - a), jnp.float32)], axis=1)
    wskip_t = W_skip.transpose(1, 0, 2)

    attr_s = _make_attr_gather(n, e, 16)(attr16, sender)
    nf, sc = _make_tc_pre(n, d, a)(node_feats, attr16, W_up, wskip_t)
    tpw_flat16 = tpw.reshape(a, r * d).astype(jnp.bfloat16)
    tpw2 = _make_tc_tpw(e, d, a, r)(edge_feats, attr_s, tpw_flat16, edge_attrs)
    partial = _make_edge_pass(n, e, d)(nf, tpw2, sender, receiver)
    real, imag = _make_tc_fin(n, d)(partial, W_lin)
    message = jnp.stack((real, imag), axis=-1).reshape(n, d, 1, 2)
    return (message, sc)


# speedup vs baseline: 3.2694x; 1.1276x over previous
import functools
import math

import jax
import jax.numpy as jnp
from jax import lax
from jax.experimental import pallas as pl
from jax.experimental.pallas import tpu as pltpu
from jax.experimental.pallas import tpu_sc as plsc

NC = 2
NS = 16
L = 16

AVG_NEIGHBORS = 32.0


def _make_attr_gather(n, e, ap):
    ce = 80
    ring = 4
    epw = e // (NC * NS)
    nchunks = epw // ce
    assert epw % ce == 0 and ce % 8 == 0
    niter = -(-nchunks // ring) * ring
    mesh = plsc.VectorSubcoreMesh(
        core_axis_name="c", subcore_axis_name="s", num_cores=NC, num_subcores=NS)

    scratch = (
        [pltpu.VMEM((ce,), jnp.int32)] * ring
        + [pltpu.VMEM((ce, ap), jnp.float32)] * ring
        + [pltpu.SemaphoreType.DMA] * (3 * ring)
    )

    @functools.partial(
        pl.kernel, mesh=mesh,
        out_type=jax.ShapeDtypeStruct((e, ap), jnp.float32),
        compiler_params=pltpu.CompilerParams(use_tc_tiling_on_sc=False),
        scratch_types=scratch,
    )
    def attr_gather(attr_hbm, sidx_hbm, out_hbm, *sc):
        sidx_v = sc[0:ring]
        rows_v = sc[ring:2 * ring]
        sem_idx = sc[2 * ring:3 * ring]
        sem_g = sc[3 * ring:4 * ring]
        sem_wb = sc[4 * ring:5 * ring]
        c = lax.axis_index("c")
        s = lax.axis_index("s")
        base0 = (c * NS + s) * epw

        def start_idx(j, slot):
            pltpu.async_copy(sidx_hbm.at[pl.ds(base0 + j * ce, ce)],
                             sidx_v[slot], sem_idx[slot])

        def wait_idx(slot):
            pltpu.make_async_copy(sidx_hbm.at[pl.ds(0, ce)], sidx_v[slot],
                                  sem_idx[slot]).wait()

        def start_gather(slot):
            pltpu.async_copy(attr_hbm.at[sidx_v[slot]], rows_v[slot],
                             sem_g[slot])

        def wait_gather(slot):
            pltpu.make_async_copy(attr_hbm.at[sidx_v[slot]], rows_v[slot],
                                  sem_g[slot]).wait()

        def start_wb(j, slot):
            pltpu.async_copy(rows_v[slot],
                             out_hbm.at[pl.ds(base0 + j * ce, ce), :],
                             sem_wb[slot])

        def wait_wb(slot):
            pltpu.make_async_copy(rows_v[slot],
                                  out_hbm.at[pl.ds(0, ce), :],
                                  sem_wb[slot]).wait()

        start_idx(0, 0)
        start_idx(1, 1)
        wait_idx(0)
        start_gather(0)

        def chunk(j4, carry):
            for b in range(ring):
                j = j4 * ring + b
                @pl.when(jnp.logical_and(j >= 2, j + 2 <= nchunks - 1))
                def _():
                    wait_wb((b + 2) % ring)
                @pl.when(j + 2 <= nchunks - 1)
                def _():
                    start_idx(j + 2, (b + 2) % ring)
                @pl.when(j + 1 <= nchunks - 1)
                def _():
                    wait_idx((b + 1) % ring)
                    start_gather((b + 1) % ring)
                @pl.when(j <= nchunks - 1)
                def _():
                    wait_gather(b)
                    start_wb(j, b)
            return carry

        lax.fori_loop(0, niter // ring, chunk, 0)
        for jj in range(nchunks - ring, nchunks):
            wait_wb(jj % ring)

    return attr_gather


def _make_edge_pass(n, e, d, seg, nseg):
    ce = 40
    ring = 4
    eseg = e // nseg
    eps = eseg // NS
    nchunks = eps // ce
    niter = -(-nchunks // ring) * ring
    assert eps % ce == 0 and nchunks >= 2 * ring
    npad = 10240
    rows_per_tile = npad // NS
    assert n <= npad and rows_per_tile % ce == 0
    nk = d // L
    mesh = plsc.VectorSubcoreMesh(
        core_axis_name="c", subcore_axis_name="s", num_cores=NC, num_subcores=NS)

    scratch = (
        [pltpu.VMEM((ce,), jnp.int32)] * ring
        + [pltpu.VMEM((ce,), jnp.int32)] * ring
        + [pltpu.VMEM((ce, d), jnp.float32)] * ring
        + [pltpu.VMEM((ce, d), jnp.float32)] * ring
        + [pltpu.VMEM_SHARED((npad, d), jnp.float32)]
        + [pltpu.SemaphoreType.DMA] * (4 * ring)
    )

    @functools.partial(
        pl.kernel, mesh=mesh,
        out_type=jax.ShapeDtypeStruct((NC, npad, d), jnp.float32),
        scratch_types=scratch,
    )
    def edge_pass(nf_hbm, tpw2_hbm, sidx_hbm, ridx_hbm, out_hbm, *sc):
        sidx_v = sc[0:ring]
        ridx_v = sc[ring:2 * ring]
        tpw_v = sc[2 * ring:3 * ring]
        nfr_v = sc[3 * ring:4 * ring]
        acc_sh = sc[4 * ring]
        sem_idx = sc[4 * ring + 1:4 * ring + 1 + ring]
        sem_tpw = sc[4 * ring + 1 + ring:4 * ring + 1 + 2 * ring]
        sem_g = sc[4 * ring + 1 + 2 * ring:4 * ring + 1 + 3 * ring]
        sem_sc = sc[4 * ring + 1 + 3 * ring:4 * ring + 1 + 4 * ring]
        c = lax.axis_index("c")
        s = lax.axis_index("s")

        def zfill(rr, carry):
            for k in range(nk):
                tpw_v[0][rr, pl.ds(k * L, L)] = jnp.zeros((L,), jnp.float32)
            return carry
        lax.fori_loop(0, ce, zfill, 0)

        def zcopy(t, carry):
            pltpu.sync_copy(
                tpw_v[0], acc_sh.at[pl.ds(s * rows_per_tile + t * ce, ce), :])
            return carry
        lax.fori_loop(0, rows_per_tile // ce, zcopy, 0)
        plsc.subcore_barrier()

        base0 = seg * eseg + s * eps

        def start_idx(j, slot):
            base = base0 + j * ce
            pltpu.async_copy(sidx_hbm.at[pl.ds(base, ce)], sidx_v[slot],
                             sem_idx[slot])
            pltpu.async_copy(ridx_hbm.at[pl.ds(base, ce)], ridx_v[slot],
                             sem_idx[slot])

        def wait_idx(slot):
            pltpu.make_async_copy(sidx_hbm.at[pl.ds(0, ce)], sidx_v[slot],
                                  sem_idx[slot]).wait()
            pltpu.make_async_copy(ridx_hbm.at[pl.ds(0, ce)], ridx_v[slot],
                                  sem_idx[slot]).wait()

        def start_data(j, slot):
            base = base0 + j * ce
            pltpu.async_copy(
                tpw2_hbm.at[c, pl.ds(base - seg * eseg, ce), :], tpw_v[slot],
                sem_tpw[slot])
            pltpu.async_copy(nf_hbm.at[sidx_v[slot]], nfr_v[slot], sem_g[slot])

        def wait_data(slot):
            pltpu.make_async_copy(tpw2_hbm.at[0, pl.ds(0, ce), :], tpw_v[slot],
                                  sem_tpw[slot]).wait()
            pltpu.make_async_copy(nf_hbm.at[sidx_v[slot]], nfr_v[slot],
                                  sem_g[slot]).wait()

        def start_scatter(slot):
            pltpu.async_copy(tpw_v[slot], acc_sh.at[ridx_v[slot]],
                             sem_sc[slot], add=True)

        def wait_scatter(slot):
            pltpu.make_async_copy(tpw_v[slot], acc_sh.at[ridx_v[slot]],
                                  sem_sc[slot]).wait()

        start_idx(0, 0)
        start_idx(1, 1)
        wait_idx(0)
        start_data(0, 0)

        def chunk(j5, carry):
            for b in range(ring):
                j = j5 * ring + b
                @pl.when(jnp.logical_and(j >= ring - 2, j <= nchunks - 3))
                def _():
                    wait_scatter((b + 2) % ring)
                @pl.when(j <= nchunks - 3)
                def _():
                    start_idx(j + 2, (b + 2) % ring)
                @pl.when(j <= nchunks - 2)
                def _():
                    wait_idx((b + 1) % ring)
                    start_data(j + 1, (b + 1) % ring)
                @pl.when(j <= nchunks - 1)
                def _():
                    wait_data(b)

                    def edge(ei, carry2):
                        for k in range(nk):
                            sl = pl.ds(k * L, L)
                            tpw_v[b][ei, sl] = (
                                tpw_v[b][ei, sl] * nfr_v[b][ei, sl])
                        return carry2
                    lax.fori_loop(0, ce, edge, 0)
                    start_scatter(b)
            return carry

        lax.fori_loop(0, niter // ring, chunk, 0)
        for jj in range(nchunks - ring, nchunks):
            wait_scatter(jj % ring)
        plsc.subcore_barrier()

        def flush(t, carry):
            r0 = s * rows_per_tile + t * ce
            pltpu.sync_copy(
                acc_sh.at[pl.ds(r0, ce), :], out_hbm.at[c, pl.ds(r0, ce), :])
            return carry
        lax.fori_loop(0, rows_per_tile // ce, flush, 0)

    return edge_pass


def _make_tc_pre(n, d, a):
    bn = 1000
    assert n % bn == 0

    def body(feats_ref, attr_ref, wup_ref, wskip_ref, nf_ref, sc_ref):
        f = feats_ref[...]
        at = attr_ref[...]
        nf_ref[...] = jnp.dot(
            f, wup_ref[...], precision=lax.Precision.HIGHEST,
            preferred_element_type=jnp.float32) * (1.0 / math.sqrt(d))
        acc = jnp.zeros((bn, d), jnp.float32)
        for ai in range(a):
            acc += at[:, ai:ai + 1] * jnp.dot(
                f, wskip_ref[ai], precision=lax.Precision.HIGHEST,
                preferred_element_type=jnp.float32)
        sc_ref[...] = acc * (1.0 / math.sqrt(float(d * a)))

    return pl.pallas_call(
        body,
        grid=(n // bn,),
        in_specs=[
            pl.BlockSpec((bn, d), lambda i: (i, 0)),
            pl.BlockSpec((bn, 16), lambda i: (i, 0)),
            pl.BlockSpec((d, d), lambda i: (0, 0)),
            pl.BlockSpec((a, d, d), lambda i: (0, 0, 0)),
        ],
        out_specs=[
            pl.BlockSpec((bn, d), lambda i: (i, 0)),
            pl.BlockSpec((bn, d), lambda i: (i, 0)),
        ],
        out_shape=[
            jax.ShapeDtypeStruct((n, d), jnp.float32),
            jax.ShapeDtypeStruct((n, d), jnp.float32),
        ],
    )


def _make_tc_tpw(e, d, a, r, seg, nseg):
    be = 2000
    eseg = e // nseg
    off = seg * eseg // be
    assert eseg % be == 0

    def body(ef_ref, attrs_ref, tpw_ref, ea_ref, out_ref):
        ef = ef_ref[...]
        at = attrs_ref[...]
        z = jnp.dot(at[:, :a].astype(jnp.bfloat16), tpw_ref[...],
                    preferred_element_type=jnp.float32)
        acc = jnp.zeros((be, d), jnp.float32)
        for ri in range(r):
            acc += ef[:, ri:ri + 1] * z[:, ri * d:(ri + 1) * d]
        ea = ea_ref[...]
        out_ref[0] = acc * ea[:, 0:1]
        out_ref[1] = acc * ea[:, 1:2]

    return pl.pallas_call(
        body,
        grid=(eseg // be,),
        in_specs=[
            pl.BlockSpec((be, r), lambda i: (i + off, 0)),
            pl.BlockSpec((be, 16), lambda i: (i + off, 0)),
            pl.BlockSpec((a, r * d), lambda i: (0, 0)),
            pl.BlockSpec((be, 2), lambda i: (i + off, 0)),
        ],
        out_specs=pl.BlockSpec((2, be, d), lambda i: (0, i, 0)),
        out_shape=jax.ShapeDtypeStruct((2, eseg, d), jnp.float32),
    )


def _make_tc_fin(n, d):
    bn = 1000
    cst = 1.0 / math.sqrt(float(d)) / AVG_NEIGHBORS

    def body(pre0_ref, pre1_ref, wlin_ref, outr_ref, outi_ref):
        w = wlin_ref[...]
        outr_ref[...] = jnp.dot(
            pre0_ref[0] + pre1_ref[0], w, precision=lax.Precision.HIGHEST,
            preferred_element_type=jnp.float32) * cst
        outi_ref[...] = jnp.dot(
            pre0_ref[1] + pre1_ref[1], w, precision=lax.Precision.HIGHEST,
            preferred_element_type=jnp.float32) * cst

    return pl.pallas_call(
        body,
        grid=(n // bn,),
        in_specs=[
            pl.BlockSpec((2, bn, d), lambda i: (0, i, 0)),
            pl.BlockSpec((2, bn, d), lambda i: (0, i, 0)),
            pl.BlockSpec((d, d), lambda i: (0, 0)),
        ],
        out_specs=[
            pl.BlockSpec((bn, d), lambda i: (i, 0)),
            pl.BlockSpec((bn, d), lambda i: (i, 0)),
        ],
        out_shape=[
            jax.ShapeDtypeStruct((n, d), jnp.float32),
            jax.ShapeDtypeStruct((n, d), jnp.float32),
        ],
    )


def kernel(node_attrs, node_feats, edge_attrs, edge_feats, edge_index,
           W_up, tpw, W_lin, W_skip):
    n, a = node_attrs.shape
    d = node_feats.shape[1]
    e, r = edge_feats.shape

    sender = edge_index[0].astype(jnp.int32)
    receiver = edge_index[1].astype(jnp.int32)
    attr16 = jnp.concatenate(
        [node_attrs, jnp.zeros((n, 16 - a), jnp.float32)], axis=1)
    wskip_t = W_skip.transpose(1, 0, 2)

    attr_s = _make_attr_gather(n, e, 16)(attr16, sender)
    nf, sc = _make_tc_pre(n, d, a)(node_feats, attr16, W_up, wskip_t)
    tpw_flat16 = tpw.reshape(a, r * d).astype(jnp.bfloat16)
    tpw2_0 = _make_tc_tpw(e, d, a, r, 0, 2)(
        edge_feats, attr_s, tpw_flat16, edge_attrs)
    tpw2_1 = _make_tc_tpw(e, d, a, r, 1, 2)(
        edge_feats, attr_s, tpw_flat16, edge_attrs)
    partial0 = _make_edge_pass(n, e, d, 0, 2)(nf, tpw2_0, sender, receiver)
    partial1 = _make_edge_pass(n, e, d, 1, 2)(nf, tpw2_1, sender, receiver)
    real, imag = _make_tc_fin(n, d)(partial0, partial1, W_lin)
    message = jnp.stack((real, imag), axis=-1).reshape(n, d, 1, 2)
    return (message, sc)


# speedup vs baseline: 3.3971x; 1.0390x over previous
import functools
import math

import jax
import jax.numpy as jnp
from jax import lax
from jax.experimental import pallas as pl
from jax.experimental.pallas import tpu as pltpu
from jax.experimental.pallas import tpu_sc as plsc

NC = 2
NS = 16
L = 16

AVG_NEIGHBORS = 32.0


def _make_attr_gather(n, e, ap):
    ce = 80
    ring = 4
    epw = e // (NC * NS)
    nchunks = epw // ce
    assert epw % ce == 0 and ce % 8 == 0
    niter = -(-nchunks // ring) * ring
    mesh = plsc.VectorSubcoreMesh(
        core_axis_name="c", subcore_axis_name="s", num_cores=NC, num_subcores=NS)

    scratch = (
        [pltpu.VMEM((ce,), jnp.int32)] * ring
        + [pltpu.VMEM((ce, ap), jnp.float32)] * ring
        + [pltpu.SemaphoreType.DMA] * (3 * ring)
    )

    @functools.partial(
        pl.kernel, mesh=mesh,
        out_type=jax.ShapeDtypeStruct((e, ap), jnp.float32),
        compiler_params=pltpu.CompilerParams(use_tc_tiling_on_sc=False),
        scratch_types=scratch,
    )
    def attr_gather(attr_hbm, sidx_hbm, out_hbm, *sc):
        sidx_v = sc[0:ring]
        rows_v = sc[ring:2 * ring]
        sem_idx = sc[2 * ring:3 * ring]
        sem_g = sc[3 * ring:4 * ring]
        sem_wb = sc[4 * ring:5 * ring]
        c = lax.axis_index("c")
        s = lax.axis_index("s")
        base0 = (c * NS + s) * epw

        def start_idx(j, slot):
            pltpu.async_copy(sidx_hbm.at[pl.ds(base0 + j * ce, ce)],
                             sidx_v[slot], sem_idx[slot])

        def wait_idx(slot):
            pltpu.make_async_copy(sidx_hbm.at[pl.ds(0, ce)], sidx_v[slot],
                                  sem_idx[slot]).wait()

        def start_gather(slot):
            pltpu.async_copy(attr_hbm.at[sidx_v[slot]], rows_v[slot],
                             sem_g[slot])

        def wait_gather(slot):
            pltpu.make_async_copy(attr_hbm.at[sidx_v[slot]], rows_v[slot],
                                  sem_g[slot]).wait()

        def start_wb(j, slot):
            pltpu.async_copy(rows_v[slot],
                             out_hbm.at[pl.ds(base0 + j * ce, ce), :],
                             sem_wb[slot])

        def wait_wb(slot):
            pltpu.make_async_copy(rows_v[slot],
                                  out_hbm.at[pl.ds(0, ce), :],
                                  sem_wb[slot]).wait()

        start_idx(0, 0)
        start_idx(1, 1)
        wait_idx(0)
        start_gather(0)

        def chunk(j4, carry):
            for b in range(ring):
                j = j4 * ring + b
                @pl.when(jnp.logical_and(j >= 2, j + 2 <= nchunks - 1))
                def _():
                    wait_wb((b + 2) % ring)
                @pl.when(j + 2 <= nchunks - 1)
                def _():
                    start_idx(j + 2, (b + 2) % ring)
                @pl.when(j + 1 <= nchunks - 1)
                def _():
                    wait_idx((b + 1) % ring)
                    start_gather((b + 1) % ring)
                @pl.when(j <= nchunks - 1)
                def _():
                    wait_gather(b)
                    start_wb(j, b)
            return carry

        lax.fori_loop(0, niter // ring, chunk, 0)
        for jj in range(nchunks - ring, nchunks):
            wait_wb(jj % ring)

    return attr_gather


def _make_edge_pass(n, e, d, seg, nseg):
    ce = 40
    ring = 4
    eseg = e // nseg
    eps = eseg // NS
    nchunks = eps // ce
    niter = -(-nchunks // ring) * ring
    assert eps % ce == 0 and nchunks >= 2 * ring
    npad = 10240
    rows_per_tile = npad // NS
    assert n <= npad and rows_per_tile % ce == 0
    nk = d // L
    mesh = plsc.VectorSubcoreMesh(
        core_axis_name="c", subcore_axis_name="s", num_cores=NC, num_subcores=NS)

    scratch = (
        [pltpu.VMEM((ce,), jnp.int32)] * ring
        + [pltpu.VMEM((ce,), jnp.int32)] * ring
        + [pltpu.VMEM((ce, d), jnp.float32)] * ring
        + [pltpu.VMEM((ce, d), jnp.float32)] * ring
        + [pltpu.VMEM_SHARED((npad, d), jnp.float32)]
        + [pltpu.SemaphoreType.DMA] * (4 * ring)
    )

    @functools.partial(
        pl.kernel, mesh=mesh,
        out_type=jax.ShapeDtypeStruct((NC, npad, d), jnp.float32),
        scratch_types=scratch,
    )
    def edge_pass(nf_hbm, tpw2_hbm, sidx_hbm, ridx_hbm, out_hbm, *sc):
        sidx_v = sc[0:ring]
        ridx_v = sc[ring:2 * ring]
        tpw_v = sc[2 * ring:3 * ring]
        nfr_v = sc[3 * ring:4 * ring]
        acc_sh = sc[4 * ring]
        sem_idx = sc[4 * ring + 1:4 * ring + 1 + ring]
        sem_tpw = sc[4 * ring + 1 + ring:4 * ring + 1 + 2 * ring]
        sem_g = sc[4 * ring + 1 + 2 * ring:4 * ring + 1 + 3 * ring]
        sem_sc = sc[4 * ring + 1 + 3 * ring:4 * ring + 1 + 4 * ring]
        c = lax.axis_index("c")
        s = lax.axis_index("s")

        def zfill(rr, carry):
            for k in range(nk):
                tpw_v[0][rr, pl.ds(k * L, L)] = jnp.zeros((L,), jnp.float32)
            return carry
        lax.fori_loop(0, ce, zfill, 0)

        def zcopy(t, carry):
            pltpu.sync_copy(
                tpw_v[0], acc_sh.at[pl.ds(s * rows_per_tile + t * ce, ce), :])
            return carry
        lax.fori_loop(0, rows_per_tile // ce, zcopy, 0)
        plsc.subcore_barrier()

        base0 = seg * eseg + s * eps

        def start_idx(j, slot):
            base = base0 + j * ce
            pltpu.async_copy(sidx_hbm.at[pl.ds(base, ce)], sidx_v[slot],
                             sem_idx[slot])
            pltpu.async_copy(ridx_hbm.at[pl.ds(base, ce)], ridx_v[slot],
                             sem_idx[slot])

        def wait_idx(slot):
            pltpu.make_async_copy(sidx_hbm.at[pl.ds(0, ce)], sidx_v[slot],
                                  sem_idx[slot]).wait()
            pltpu.make_async_copy(ridx_hbm.at[pl.ds(0, ce)], ridx_v[slot],
                                  sem_idx[slot]).wait()

        def start_data(j, slot):
            base = base0 + j * ce
            pltpu.async_copy(
                tpw2_hbm.at[c, pl.ds(base - seg * eseg, ce), :], tpw_v[slot],
                sem_tpw[slot])
            pltpu.async_copy(nf_hbm.at[sidx_v[slot]], nfr_v[slot], sem_g[slot])

        def wait_data(slot):
            pltpu.make_async_copy(tpw2_hbm.at[0, pl.ds(0, ce), :], tpw_v[slot],
                                  sem_tpw[slot]).wait()
            pltpu.make_async_copy(nf_hbm.at[sidx_v[slot]], nfr_v[slot],
                                  sem_g[slot]).wait()

        def start_scatter(slot):
            pltpu.async_copy(tpw_v[slot], acc_sh.at[ridx_v[slot]],
                             sem_sc[slot], add=True)

        def wait_scatter(slot):
            pltpu.make_async_copy(tpw_v[slot], acc_sh.at[ridx_v[slot]],
                                  sem_sc[slot]).wait()

        start_idx(0, 0)
        start_idx(1, 1)
        wait_idx(0)
        start_data(0, 0)

        def chunk(j5, carry):
            for b in range(ring):
                j = j5 * ring + b
                @pl.when(jnp.logical_and(j >= ring - 2, j <= nchunks - 3))
                def _():
                    wait_scatter((b + 2) % ring)
                @pl.when(j <= nchunks - 3)
                def _():
                    start_idx(j + 2, (b + 2) % ring)
                @pl.when(j <= nchunks - 2)
                def _():
                    wait_idx((b + 1) % ring)
                    start_data(j + 1, (b + 1) % ring)
                @pl.when(j <= nchunks - 1)
                def _():
                    wait_data(b)

                    def edge(e2, carry2):
                        for u in range(2):
                            ei = e2 * 2 + u
                            for k in range(nk):
                                sl = pl.ds(k * L, L)
                                tpw_v[b][ei, sl] = (
                                    tpw_v[b][ei, sl] * nfr_v[b][ei, sl])
                        return carry2
                    lax.fori_loop(0, ce // 2, edge, 0)
                    start_scatter(b)
            return carry

        lax.fori_loop(0, niter // ring, chunk, 0)
        for jj in range(nchunks - ring, nchunks):
            wait_scatter(jj % ring)
        plsc.subcore_barrier()

        def flush(t, carry):
            r0 = s * rows_per_tile + t * ce
            pltpu.sync_copy(
                acc_sh.at[pl.ds(r0, ce), :], out_hbm.at[c, pl.ds(r0, ce), :])
            return carry
        lax.fori_loop(0, rows_per_tile // ce, flush, 0)

    return edge_pass


def _make_tc_pre(n, d, a):
    bn = 1000
    assert n % bn == 0

    def body(feats_ref, attr_ref, wup_ref, wskip_ref, nf_ref, sc_ref):
        f = feats_ref[...]
        at = attr_ref[...]
        nf_ref[...] = jnp.dot(
            f, wup_ref[...], precision=lax.Precision.HIGHEST,
            preferred_element_type=jnp.float32) * (1.0 / math.sqrt(d))
        acc = jnp.zeros((bn, d), jnp.float32)
        for ai in range(a):
            acc += at[:, ai:ai + 1] * jnp.dot(
                f, wskip_ref[ai], precision=lax.Precision.HIGHEST,
                preferred_element_type=jnp.float32)
        sc_ref[...] = acc * (1.0 / math.sqrt(float(d * a)))

    return pl.pallas_call(
        body,
        grid=(n // bn,),
        in_specs=[
            pl.BlockSpec((bn, d), lambda i: (i, 0)),
            pl.BlockSpec((bn, 16), lambda i: (i, 0)),
            pl.BlockSpec((d, d), lambda i: (0, 0)),
            pl.BlockSpec((a, d, d), lambda i: (0, 0, 0)),
        ],
        out_specs=[
            pl.BlockSpec((bn, d), lambda i: (i, 0)),
            pl.BlockSpec((bn, d), lambda i: (i, 0)),
        ],
        out_shape=[
            jax.ShapeDtypeStruct((n, d), jnp.float32),
            jax.ShapeDtypeStruct((n, d), jnp.float32),
        ],
    )


def _make_tc_tpw(e, d, a, r, seg, nseg):
    be = 2000
    eseg = e // nseg
    off = seg * eseg // be
    assert eseg % be == 0

    def body(ef_ref, attrs_ref, tpw_ref, ea_ref, out_ref):
        ef = ef_ref[...]
        at = attrs_ref[...]
        z = jnp.dot(at[:, :a].astype(jnp.bfloat16), tpw_ref[...],
                    preferred_element_type=jnp.float32)
        acc = jnp.zeros((be, d), jnp.float32)
        for ri in range(r):
            acc += ef[:, ri:ri + 1] * z[:, ri * d:(ri + 1) * d]
        ea = ea_ref[...]
        out_ref[0] = acc * ea[:, 0:1]
        out_ref[1] = acc * ea[:, 1:2]

    return pl.pallas_call(
        body,
        grid=(eseg // be,),
        in_specs=[
            pl.BlockSpec((be, r), lambda i: (i + off, 0)),
            pl.BlockSpec((be, 16), lambda i: (i + off, 0)),
            pl.BlockSpec((a, r * d), lambda i: (0, 0)),
            pl.BlockSpec((be, 2), lambda i: (i + off, 0)),
        ],
        out_specs=pl.BlockSpec((2, be, d), lambda i: (0, i, 0)),
        out_shape=jax.ShapeDtypeStruct((2, eseg, d), jnp.float32),
    )


def _make_tc_fin(n, d):
    bn = 1000
    cst = 1.0 / math.sqrt(float(d)) / AVG_NEIGHBORS

    def body(pre0_ref, pre1_ref, pre2_ref, pre3_ref, wlin_ref,
             outr_ref, outi_ref):
        w = wlin_ref[...]
        pr = (pre0_ref[0] + pre1_ref[0]) + (pre2_ref[0] + pre3_ref[0])
        pi = (pre0_ref[1] + pre1_ref[1]) + (pre2_ref[1] + pre3_ref[1])
        outr_ref[...] = jnp.dot(
            pr, w, precision=lax.Precision.HIGHEST,
            preferred_element_type=jnp.float32) * cst
        outi_ref[...] = jnp.dot(
            pi, w, precision=lax.Precision.HIGHEST,
            preferred_element_type=jnp.float32) * cst

    return pl.pallas_call(
        body,
        grid=(n // bn,),
        in_specs=[
            pl.BlockSpec((2, bn, d), lambda i: (0, i, 0)),
            pl.BlockSpec((2, bn, d), lambda i: (0, i, 0)),
            pl.BlockSpec((2, bn, d), lambda i: (0, i, 0)),
            pl.BlockSpec((2, bn, d), lambda i: (0, i, 0)),
            pl.BlockSpec((d, d), lambda i: (0, 0)),
        ],
        out_specs=[
            pl.BlockSpec((bn, d), lambda i: (i, 0)),
            pl.BlockSpec((bn, d), lambda i: (i, 0)),
        ],
        out_shape=[
            jax.ShapeDtypeStruct((n, d), jnp.float32),
            jax.ShapeDtypeStruct((n, d), jnp.float32),
        ],
    )


def kernel(node_attrs, node_feats, edge_attrs, edge_feats, edge_index,
           W_up, tpw, W_lin, W_skip):
    n, a = node_attrs.shape
    d = node_feats.shape[1]
    e, r = edge_feats.shape

    sender = edge_index[0].astype(jnp.int32)
    receiver = edge_index[1].astype(jnp.int32)
    attr16 = jnp.concatenate(
        [node_attrs, jnp.zeros((n, 16 - a), jnp.float32)], axis=1)
    wskip_t = W_skip.transpose(1, 0, 2)

    attr_s = _make_attr_gather(n, e, 16)(attr16, sender)
    nf, sc = _make_tc_pre(n, d, a)(node_feats, attr16, W_up, wskip_t)
    tpw_flat16 = tpw.reshape(a, r * d).astype(jnp.bfloat16)
    parts = []
    for seg in range(4):
        tpw2_s = _make_tc_tpw(e, d, a, r, seg, 4)(
            edge_feats, attr_s, tpw_flat16, edge_attrs)
        parts.append(
            _make_edge_pass(n, e, d, seg, 4)(nf, tpw2_s, sender, receiver))
    real, imag = _make_tc_fin(n, d)(*parts, W_lin)
    message = jnp.stack((real, imag), axis=-1).reshape(n, d, 1, 2)
    return (message, sc)
